# baseline (speedup 1.0000x reference)
import functools

import jax
import jax.numpy as jnp
from jax import lax
from jax.experimental import pallas as pl
from jax.experimental.pallas import tpu as pltpu
from jax.experimental.pallas import tpu_sc as plsc


def _make_gather(S, B0, V, D):
    L = 128
    NTB = B0 // L
    TD = D // 8
    n_chunks = S * NTB
    info = plsc.get_sparse_core_info()
    NW = info.num_cores * info.num_subcores
    c_per_w = n_chunks // NW
    NB = 4
    n_groups = c_per_w // NB

    mesh = plsc.VectorSubcoreMesh(core_axis_name="c", subcore_axis_name="s")

    @functools.partial(
        pl.kernel,
        mesh=mesh,
        out_type=jax.ShapeDtypeStruct((S, TD, NTB, 8 * L), jnp.float32),
        scratch_types=[
            [pltpu.VMEM((L,), jnp.int32) for _ in range(NB)],
            [pltpu.VMEM((L, D), jnp.float32) for _ in range(NB)],
            [pltpu.VMEM((TD, 8 * L), jnp.float32) for _ in range(NB)],
            [pltpu.SemaphoreType.DMA for _ in range(NB)],
            [pltpu.SemaphoreType.DMA for _ in range(NB)],
            [pltpu.SemaphoreType.DMA for _ in range(NB)],
        ],
        compiler_params=pltpu.CompilerParams(
            use_tc_tiling_on_sc=False, needs_layout_passes=False),
    )
    def gather_kernel(idx_hbm, table_hbm, out_hbm, idx_v, rows_v, tile_v,
                      sem_i, sem_g, sem_o):
        wid = lax.axis_index("s") * info.num_cores + lax.axis_index("c")
        cbase = wid * c_per_w
        iota16 = lax.iota(jnp.int32, 16)

        def idx_src(c):
            s, tb = c // NTB, lax.rem(c, NTB)
            return idx_hbm.at[s, pl.ds(tb * L, L)]

        def out_dst(c):
            s, tb = c // NTB, lax.rem(c, NTB)
            return out_hbm.at[s, :, tb]

        for b in range(NB):
            pltpu.async_copy(idx_src(cbase + b), idx_v[b], sem_i[b])

        def transpose_chunk(b):
            def seg_body(seg, carry):
                off = seg * 16
                bc = iota16 + off
                for d in range(D):
                    v = plsc.load_gather(rows_v[b], [bc, iota16 * 0 + d])
                    tile_v[b][d // 8, pl.ds((d % 8) * L + off, 16)] = v
                return carry

            lax.fori_loop(0, L // 16, seg_body, 0)

        def body(g, carry):
            g0 = cbase + g * NB
            for b in range(NB):
                pltpu.make_async_copy(idx_src(g0 + b), idx_v[b],
                                      sem_i[b]).wait()
                pltpu.async_copy(table_hbm.at[idx_v[b]], rows_v[b], sem_g[b])
            for b in range(NB):
                c = g0 + b
                pltpu.make_async_copy(table_hbm.at[idx_v[b]], rows_v[b],
                                      sem_g[b]).wait()
                nc = jnp.minimum(c + NB, n_chunks - 1)
                pltpu.async_copy(idx_src(nc), idx_v[b], sem_i[b])

                @pl.when(g != 0)
                def _():
                    pltpu.make_async_copy(tile_v[b], out_dst(c - NB),
                                          sem_o[b]).wait()

                transpose_chunk(b)
                pltpu.async_copy(tile_v[b], out_dst(c), sem_o[b])
            return carry

        lax.fori_loop(0, n_groups, body, 0)

        for b in range(NB):
            c = cbase + (n_groups - 1) * NB + b
            pltpu.make_async_copy(tile_v[b], out_dst(c), sem_o[b]).wait()
            pltpu.make_async_copy(idx_src(c), idx_v[b], sem_i[b]).wait()

    return gather_kernel


def kernel(data, table):
    B0, S = data.shape
    V, D = table.shape
    out5 = _make_gather(S, B0, V, D)(data.T, table)
    out5 = out5.reshape(S, D // 8, B0 // 128, 8, 128)
    return out5.transpose((2, 4, 0, 1, 3)).reshape(B0, S, D)


# speedup vs baseline: 1.4223x; 1.4223x over previous
import functools

import jax
import jax.numpy as jnp
from jax import lax
from jax.experimental import pallas as pl
from jax.experimental.pallas import tpu as pltpu
from jax.experimental.pallas import tpu_sc as plsc


def _make_gather(S, B0, V, D):
    L = 128
    NTB = B0 // L
    TD = D // 8
    n_chunks = S * NTB
    info = plsc.get_sparse_core_info()
    NW = info.num_cores * info.num_subcores
    c_per_w = n_chunks // NW
    NB = 4
    n_groups = c_per_w // NB

    mesh = plsc.VectorSubcoreMesh(core_axis_name="c", subcore_axis_name="s")

    @functools.partial(
        pl.kernel,
        mesh=mesh,
        out_type=jax.ShapeDtypeStruct((S, TD, NTB, 8 * L), jnp.float32),
        scratch_types=[
            [pltpu.VMEM((L,), jnp.int32) for _ in range(NB)],
            [pltpu.VMEM((L, D), jnp.float32) for _ in range(NB)],
            [pltpu.VMEM((TD, 8 * L), jnp.float32) for _ in range(NB)],
            [pltpu.SemaphoreType.DMA for _ in range(NB)],
            [pltpu.SemaphoreType.DMA for _ in range(NB)],
            [pltpu.SemaphoreType.DMA for _ in range(NB)],
        ],
        compiler_params=pltpu.CompilerParams(
            use_tc_tiling_on_sc=False, needs_layout_passes=False),
    )
    def gather_kernel(idx_hbm, table_hbm, out_hbm, idx_v, rows_v, tile_v,
                      sem_i, sem_g, sem_o):
        wid = lax.axis_index("s") * info.num_cores + lax.axis_index("c")
        cbase = wid * c_per_w
        iota16 = lax.iota(jnp.int32, 16)

        def idx_src(c):
            s, tb = c // NTB, lax.rem(c, NTB)
            return idx_hbm.at[s, pl.ds(tb * L, L)]

        def out_dst(c):
            s, tb = c // NTB, lax.rem(c, NTB)
            return out_hbm.at[s, :, tb]

        for b in range(NB):
            pltpu.async_copy(idx_src(cbase + b), idx_v[b], sem_i[b])

        def transpose_chunk(b):
            @plsc.parallel_loop(0, (L // 16) * D, unroll=8)
            def tbody(i):
                d = i & (D - 1)
                off = (i // D) * 16
                bc = iota16 + off
                dvec = jnp.full((16,), 0, jnp.int32) + d
                v = plsc.load_gather(rows_v[b], [bc, dvec])
                tile_v[b][d // 8, pl.ds((d % 8) * L + off, 16)] = v

        def body(g, carry):
            g0 = cbase + g * NB
            for b in range(NB):
                pltpu.make_async_copy(idx_src(g0 + b), idx_v[b],
                                      sem_i[b]).wait()
                pltpu.async_copy(table_hbm.at[idx_v[b]], rows_v[b], sem_g[b])
            for b in range(NB):
                c = g0 + b
                pltpu.make_async_copy(table_hbm.at[idx_v[b]], rows_v[b],
                                      sem_g[b]).wait()
                nc = jnp.minimum(c + NB, n_chunks - 1)
                pltpu.async_copy(idx_src(nc), idx_v[b], sem_i[b])

                @pl.when(g != 0)
                def _():
                    pltpu.make_async_copy(tile_v[b], out_dst(c - NB),
                                          sem_o[b]).wait()

                transpose_chunk(b)
                pltpu.async_copy(tile_v[b], out_dst(c), sem_o[b])
            return carry

        lax.fori_loop(0, n_groups, body, 0)

        for b in range(NB):
            c = cbase + (n_groups - 1) * NB + b
            pltpu.make_async_copy(tile_v[b], out_dst(c), sem_o[b]).wait()
            pltpu.make_async_copy(idx_src(c), idx_v[b], sem_i[b]).wait()

    return gather_kernel


def kernel(data, table):
    B0, S = data.shape
    V, D = table.shape
    out5 = _make_gather(S, B0, V, D)(data.T, table)
    out5 = out5.reshape(S, D // 8, B0 // 128, 8, 128)
    return out5.transpose((2, 4, 0, 1, 3)).reshape(B0, S, D)
